# Initial kernel scaffold; baseline (speedup 1.0000x reference)
import jax
import jax.numpy as jnp
from jax.experimental import pallas as pl


def kernel(input_, hidden, batch_sizes, W_ih, W_hh, b_ih, b_hh):
    raise NotImplementedError("write your pallas kernel here")


# speedup vs baseline: 10.2966x; 10.2966x over previous
import functools

import jax
import jax.numpy as jnp
from jax.experimental import pallas as pl
from jax.experimental.pallas import tpu as pltpu


def _gi_matmul_kernel(x_ref, w_ref, b_ref, o_ref):
    o_ref[...] = (
        jnp.dot(x_ref[...], w_ref[...], preferred_element_type=jnp.float32)
        + b_ref[...]
    )


def _recurrent_kernel(gi_ref, w_ref, b_ref, h0_ref, o_ref, h_ref, *, steps, H):
    @pl.when(pl.program_id(0) == 0)
    def _init():
        h_ref[...] = h0_ref[...]

    def step(i, h):
        gh = (
            jnp.dot(h, w_ref[...], preferred_element_type=jnp.float32)
            + b_ref[...]
        )
        gi = gi_ref[pl.ds(i, 1), :]
        r = jax.nn.sigmoid(gi[:, :H] + gh[:, :H])
        z = jax.nn.sigmoid(gi[:, H : 2 * H] + gh[:, H : 2 * H])
        n = jnp.tanh(gi[:, 2 * H :] + r * gh[:, 2 * H :])
        h_new = (1.0 - z) * n + z * h
        o_ref[pl.ds(i, 1), :] = h_new
        return h_new

    h_ref[...] = jax.lax.fori_loop(0, steps, step, h_ref[...])


def kernel(input_, hidden, batch_sizes, W_ih, W_hh, b_ih, b_hh):
    del batch_sizes
    T, D = input_.shape
    H = hidden.shape[1]
    G = 3 * H

    w_ih_t = W_ih.T.astype(jnp.float32)
    w_hh_t = W_hh.T.astype(jnp.float32)
    b_ih_2d = b_ih.reshape(1, G)
    b_hh_2d = b_hh.reshape(1, G)

    TM, TN = 256, 1024
    gi = pl.pallas_call(
        _gi_matmul_kernel,
        grid=(G // TN, T // TM),
        in_specs=[
            pl.BlockSpec((TM, D), lambda j, i: (i, 0)),
            pl.BlockSpec((D, TN), lambda j, i: (0, j)),
            pl.BlockSpec((1, TN), lambda j, i: (0, j)),
        ],
        out_specs=pl.BlockSpec((TM, TN), lambda j, i: (i, j)),
        out_shape=jax.ShapeDtypeStruct((T, G), jnp.float32),
    )(input_, w_ih_t, b_ih_2d)

    S = 16
    out = pl.pallas_call(
        functools.partial(_recurrent_kernel, steps=S, H=H),
        grid=(T // S,),
        in_specs=[
            pl.BlockSpec((S, G), lambda i: (i, 0)),
            pl.BlockSpec((H, G), lambda i: (0, 0)),
            pl.BlockSpec((1, G), lambda i: (0, 0)),
            pl.BlockSpec((1, H), lambda i: (0, 0)),
        ],
        out_specs=pl.BlockSpec((S, H), lambda i: (i, 0)),
        out_shape=jax.ShapeDtypeStruct((T, H), jnp.float32),
        scratch_shapes=[pltpu.VMEM((1, H), jnp.float32)],
    )(gi, w_hh_t, b_hh_2d, hidden)

    final_hidden = jax.lax.slice_in_dim(out, T - 1, T, axis=0)
    return (final_hidden, out)


# speedup vs baseline: 10.3658x; 1.0067x over previous
import functools

import jax
import jax.numpy as jnp
from jax.experimental import pallas as pl
from jax.experimental.pallas import tpu as pltpu


def _gi_matmul_kernel(x_ref, w_ref, b_ref, o_ref):
    o_ref[...] = (
        jnp.dot(x_ref[...], w_ref[...], preferred_element_type=jnp.float32)
        + b_ref[...]
    )


def _recurrent_kernel(gi_ref, w_ref, b_ref, h0_ref, o_ref, h_ref, *, steps, H):
    @pl.when(pl.program_id(0) == 0)
    def _init():
        h_ref[...] = h0_ref[...]

    def step(i, h):
        gh = (
            jnp.dot(
                h.astype(jnp.bfloat16),
                w_ref[...],
                preferred_element_type=jnp.float32,
            )
            + b_ref[...]
        )
        gi = gi_ref[pl.ds(i, 1), :]
        r = jax.nn.sigmoid(gi[:, :H] + gh[:, :H])
        z = jax.nn.sigmoid(gi[:, H : 2 * H] + gh[:, H : 2 * H])
        n = jnp.tanh(gi[:, 2 * H :] + r * gh[:, 2 * H :])
        h_new = (1.0 - z) * n + z * h
        o_ref[pl.ds(i, 1), :] = h_new
        return h_new

    h_ref[...] = jax.lax.fori_loop(0, steps, step, h_ref[...])


def kernel(input_, hidden, batch_sizes, W_ih, W_hh, b_ih, b_hh):
    del batch_sizes
    T, D = input_.shape
    H = hidden.shape[1]
    G = 3 * H

    w_ih_t = W_ih.T.astype(jnp.float32)
    w_hh_t = W_hh.T.astype(jnp.bfloat16)
    b_ih_2d = b_ih.reshape(1, G)
    b_hh_2d = b_hh.reshape(1, G)

    TM, TN = 256, 1024
    gi = pl.pallas_call(
        _gi_matmul_kernel,
        grid=(G // TN, T // TM),
        in_specs=[
            pl.BlockSpec((TM, D), lambda j, i: (i, 0)),
            pl.BlockSpec((D, TN), lambda j, i: (0, j)),
            pl.BlockSpec((1, TN), lambda j, i: (0, j)),
        ],
        out_specs=pl.BlockSpec((TM, TN), lambda j, i: (i, j)),
        out_shape=jax.ShapeDtypeStruct((T, G), jnp.float32),
    )(input_, w_ih_t, b_ih_2d)

    S = 16
    out = pl.pallas_call(
        functools.partial(_recurrent_kernel, steps=S, H=H),
        grid=(T // S,),
        in_specs=[
            pl.BlockSpec((S, G), lambda i: (i, 0)),
            pl.BlockSpec((H, G), lambda i: (0, 0)),
            pl.BlockSpec((1, G), lambda i: (0, 0)),
            pl.BlockSpec((1, H), lambda i: (0, 0)),
        ],
        out_specs=pl.BlockSpec((S, H), lambda i: (i, 0)),
        out_shape=jax.ShapeDtypeStruct((T, H), jnp.float32),
        scratch_shapes=[pltpu.VMEM((1, H), jnp.float32)],
    )(gi, w_hh_t, b_hh_2d, hidden)

    final_hidden = jax.lax.slice_in_dim(out, T - 1, T, axis=0)
    return (final_hidden, out)


# speedup vs baseline: 10.5313x; 1.0160x over previous
import functools

import jax
import jax.numpy as jnp
from jax.experimental import pallas as pl
from jax.experimental.pallas import tpu as pltpu


def _gi_matmul_kernel(x_ref, w_ref, b_ref, o_ref):
    o_ref[...] = (
        jnp.dot(x_ref[...], w_ref[...], preferred_element_type=jnp.float32)
        + b_ref[...]
    )


def _recurrent_kernel(gi_ref, w_ref, b_ref, h0_ref, o_ref, h_ref, *, steps, H):
    @pl.when(pl.program_id(0) == 0)
    def _init():
        h_ref[...] = h0_ref[...]

    def step(i, h):
        h16 = h.astype(jnp.bfloat16)
        gi = gi_ref[pl.ds(i, 1), :]
        g_rz = jnp.dot(
            h16, w_ref[:, : 2 * H], preferred_element_type=jnp.float32
        ) + b_ref[:, : 2 * H]
        r = jax.nn.sigmoid(gi[:, :H] + g_rz[:, :H])
        z = jax.nn.sigmoid(gi[:, H : 2 * H] + g_rz[:, H:])
        g_n = jnp.dot(
            h16, w_ref[:, 2 * H :], preferred_element_type=jnp.float32
        ) + b_ref[:, 2 * H :]
        n = jnp.tanh(gi[:, 2 * H :] + r * g_n)
        h_new = (1.0 - z) * n + z * h
        o_ref[pl.ds(i, 1), :] = h_new
        return h_new

    h_ref[...] = jax.lax.fori_loop(0, steps, step, h_ref[...], unroll=2)


def kernel(input_, hidden, batch_sizes, W_ih, W_hh, b_ih, b_hh):
    del batch_sizes
    T, D = input_.shape
    H = hidden.shape[1]
    G = 3 * H

    w_ih_t = W_ih.T.astype(jnp.float32)
    w_hh_t = W_hh.T.astype(jnp.bfloat16)
    b_ih_2d = b_ih.reshape(1, G)
    b_hh_2d = b_hh.reshape(1, G)

    TM, TN = 256, 1024
    gi = pl.pallas_call(
        _gi_matmul_kernel,
        grid=(G // TN, T // TM),
        in_specs=[
            pl.BlockSpec((TM, D), lambda j, i: (i, 0)),
            pl.BlockSpec((D, TN), lambda j, i: (0, j)),
            pl.BlockSpec((1, TN), lambda j, i: (0, j)),
        ],
        out_specs=pl.BlockSpec((TM, TN), lambda j, i: (i, j)),
        out_shape=jax.ShapeDtypeStruct((T, G), jnp.float32),
    )(input_, w_ih_t, b_ih_2d)

    S = 16
    out = pl.pallas_call(
        functools.partial(_recurrent_kernel, steps=S, H=H),
        grid=(T // S,),
        in_specs=[
            pl.BlockSpec((S, G), lambda i: (i, 0)),
            pl.BlockSpec((H, G), lambda i: (0, 0)),
            pl.BlockSpec((1, G), lambda i: (0, 0)),
            pl.BlockSpec((1, H), lambda i: (0, 0)),
        ],
        out_specs=pl.BlockSpec((S, H), lambda i: (i, 0)),
        out_shape=jax.ShapeDtypeStruct((T, H), jnp.float32),
        scratch_shapes=[pltpu.VMEM((1, H), jnp.float32)],
    )(gi, w_hh_t, b_hh_2d, hidden)

    final_hidden = jax.lax.slice_in_dim(out, T - 1, T, axis=0)
    return (final_hidden, out)


# speedup vs baseline: 10.7112x; 1.0171x over previous
import functools

import jax
import jax.numpy as jnp
from jax.experimental import pallas as pl
from jax.experimental.pallas import tpu as pltpu

_NC = 4


def _gi_matmul_kernel(x_ref, w_ref, b_ref, o_ref):
    o_ref[...] = (
        jnp.dot(x_ref[...], w_ref[...], preferred_element_type=jnp.float32)
        + b_ref[...]
    )


def _recurrent_kernel(gi_ref, w_ref, b_ref, h0_ref, o_ref, h_ref, *, steps, H):
    @pl.when(pl.program_id(0) == 0)
    def _init():
        h_ref[...] = h0_ref[...]

    C = H // _NC

    def step(i, h):
        h16 = h.astype(jnp.bfloat16)
        gi = gi_ref[pl.ds(i, 1), :]
        h_parts = []
        for c in range(_NC):
            lo = 3 * C * c
            g = (
                jnp.dot(
                    h16,
                    w_ref[:, lo : lo + 3 * C],
                    preferred_element_type=jnp.float32,
                )
                + b_ref[:, lo : lo + 3 * C]
            )
            gic = gi[:, lo : lo + 3 * C]
            r = jax.nn.sigmoid(gic[:, :C] + g[:, :C])
            z = jax.nn.sigmoid(gic[:, C : 2 * C] + g[:, C : 2 * C])
            n = jnp.tanh(gic[:, 2 * C :] + r * g[:, 2 * C :])
            h_parts.append((1.0 - z) * n + z * h[:, c * C : (c + 1) * C])
        h_new = jnp.concatenate(h_parts, axis=1)
        o_ref[pl.ds(i, 1), :] = h_new
        return h_new

    h_ref[...] = jax.lax.fori_loop(0, steps, step, h_ref[...], unroll=2)


def _permute_gate_cols(w, H):
    C = H // _NC
    rows = w.shape[0]
    return (
        w.reshape(rows, 3, _NC, C)
        .transpose(0, 2, 1, 3)
        .reshape(rows, 3 * H)
    )


def kernel(input_, hidden, batch_sizes, W_ih, W_hh, b_ih, b_hh):
    del batch_sizes
    T, D = input_.shape
    H = hidden.shape[1]
    G = 3 * H

    w_ih_t = _permute_gate_cols(W_ih.T.astype(jnp.float32), H)
    w_hh_t = _permute_gate_cols(W_hh.T.astype(jnp.bfloat16), H)
    b_ih_2d = _permute_gate_cols(b_ih.reshape(1, G), H)
    b_hh_2d = _permute_gate_cols(b_hh.reshape(1, G), H)

    TM, TN = 256, 1024
    gi = pl.pallas_call(
        _gi_matmul_kernel,
        grid=(G // TN, T // TM),
        in_specs=[
            pl.BlockSpec((TM, D), lambda j, i: (i, 0)),
            pl.BlockSpec((D, TN), lambda j, i: (0, j)),
            pl.BlockSpec((1, TN), lambda j, i: (0, j)),
        ],
        out_specs=pl.BlockSpec((TM, TN), lambda j, i: (i, j)),
        out_shape=jax.ShapeDtypeStruct((T, G), jnp.float32),
    )(input_, w_ih_t, b_ih_2d)

    S = 16
    out = pl.pallas_call(
        functools.partial(_recurrent_kernel, steps=S, H=H),
        grid=(T // S,),
        in_specs=[
            pl.BlockSpec((S, G), lambda i: (i, 0)),
            pl.BlockSpec((H, G), lambda i: (0, 0)),
            pl.BlockSpec((1, G), lambda i: (0, 0)),
            pl.BlockSpec((1, H), lambda i: (0, 0)),
        ],
        out_specs=pl.BlockSpec((S, H), lambda i: (i, 0)),
        out_shape=jax.ShapeDtypeStruct((T, H), jnp.float32),
        scratch_shapes=[pltpu.VMEM((1, H), jnp.float32)],
    )(gi, w_hh_t, b_hh_2d, hidden)

    final_hidden = jax.lax.slice_in_dim(out, T - 1, T, axis=0)
    return (final_hidden, out)


# speedup vs baseline: 11.3600x; 1.0606x over previous
import functools

import jax
import jax.numpy as jnp
from jax.experimental import pallas as pl
from jax.experimental.pallas import tpu as pltpu

_NC = 4


def _gi_matmul_kernel(x_ref, w_ref, b_ref, o_ref):
    o_ref[...] = (
        jnp.dot(x_ref[...], w_ref[...], preferred_element_type=jnp.float32)
        + b_ref[...]
    ).astype(jnp.bfloat16)


def _recurrent_kernel(gi_ref, w_ref, b_ref, h0_ref, o_ref, h_ref, *, steps, H):
    @pl.when(pl.program_id(0) == 0)
    def _init():
        h_ref[...] = h0_ref[...]

    C = H // _NC

    def group(j, h):
        gi8 = gi_ref[pl.ds(j * 8, 8), :].astype(jnp.float32)
        for k in range(8):
            h16 = h.astype(jnp.bfloat16)
            gi = gi8[k : k + 1, :]
            h_parts = []
            for c in range(_NC):
                lo = 3 * C * c
                g = (
                    jnp.dot(
                        h16,
                        w_ref[:, lo : lo + 3 * C],
                        preferred_element_type=jnp.float32,
                    )
                    + b_ref[:, lo : lo + 3 * C]
                )
                gic = gi[:, lo : lo + 3 * C]
                r = jax.nn.sigmoid(gic[:, :C] + g[:, :C])
                z = jax.nn.sigmoid(gic[:, C : 2 * C] + g[:, C : 2 * C])
                n = jnp.tanh(gic[:, 2 * C :] + r * g[:, 2 * C :])
                h_parts.append((1.0 - z) * n + z * h[:, c * C : (c + 1) * C])
            h = jnp.concatenate(h_parts, axis=1)
            o_ref[pl.ds(j * 8 + k, 1), :] = h
        return h

    h_ref[...] = jax.lax.fori_loop(0, steps // 8, group, h_ref[...])


def _permute_gate_cols(w, H):
    C = H // _NC
    rows = w.shape[0]
    return (
        w.reshape(rows, 3, _NC, C)
        .transpose(0, 2, 1, 3)
        .reshape(rows, 3 * H)
    )


def kernel(input_, hidden, batch_sizes, W_ih, W_hh, b_ih, b_hh):
    del batch_sizes
    T, D = input_.shape
    H = hidden.shape[1]
    G = 3 * H

    w_ih_t = _permute_gate_cols(W_ih.T.astype(jnp.float32), H)
    w_hh_t = _permute_gate_cols(W_hh.T.astype(jnp.bfloat16), H)
    b_ih_2d = _permute_gate_cols(b_ih.reshape(1, G), H)
    b_hh_2d = _permute_gate_cols(b_hh.reshape(1, G), H)

    TM, TN = 256, 1024
    gi = pl.pallas_call(
        _gi_matmul_kernel,
        grid=(G // TN, T // TM),
        in_specs=[
            pl.BlockSpec((TM, D), lambda j, i: (i, 0)),
            pl.BlockSpec((D, TN), lambda j, i: (0, j)),
            pl.BlockSpec((1, TN), lambda j, i: (0, j)),
        ],
        out_specs=pl.BlockSpec((TM, TN), lambda j, i: (i, j)),
        out_shape=jax.ShapeDtypeStruct((T, G), jnp.bfloat16),
    )(input_, w_ih_t, b_ih_2d)

    S = 16
    out = pl.pallas_call(
        functools.partial(_recurrent_kernel, steps=S, H=H),
        grid=(T // S,),
        in_specs=[
            pl.BlockSpec((S, G), lambda i: (i, 0)),
            pl.BlockSpec((H, G), lambda i: (0, 0)),
            pl.BlockSpec((1, G), lambda i: (0, 0)),
            pl.BlockSpec((1, H), lambda i: (0, 0)),
        ],
        out_specs=pl.BlockSpec((S, H), lambda i: (i, 0)),
        out_shape=jax.ShapeDtypeStruct((T, H), jnp.float32),
        scratch_shapes=[pltpu.VMEM((1, H), jnp.float32)],
    )(gi, w_hh_t, b_hh_2d, hidden)

    final_hidden = jax.lax.slice_in_dim(out, T - 1, T, axis=0)
    return (final_hidden, out)
